# baseline (speedup 1.0000x reference)
import functools

import jax
import jax.numpy as jnp
from jax import lax
from jax.experimental import pallas as pl
from jax.experimental.pallas import tpu as pltpu
from jax.experimental.pallas import tpu_sc as plsc

BATCH = 1024
N_INST = 2
D_IN = 128
D_SAE = 16384
TOPK = 64
K_TOTAL = BATCH * N_INST * TOPK
N_TOTAL = BATCH * N_INST * D_SAE
NBINS = 65536

NC = 2
NS = 16
NTILES = NC * NS
N_PER_TILE = N_TOTAL // NTILES
CHUNK = 16384
NCHUNKS = N_PER_TILE // CHUNK
ROW_W = N_INST * D_SAE
CPR = ROW_W // CHUNK
UNROLL = 8

B_BLK = 256
H_BLK = 2048
NB = BATCH // B_BLK
NH = D_SAE // H_BLK


def _enc_body(x_ref, w_ref, b_ref, out_ref):
    a = lax.dot_general(x_ref[0], w_ref[0], (((1,), (1,)), ((), ())),
                        preferred_element_type=jnp.float32)
    out_ref[...] = lax.bitcast_convert_type(jnp.maximum(a + b_ref[0], 0.0),
                                            jnp.int32)


def _encode(xT, W_enc, b_enc3):
    return pl.pallas_call(
        _enc_body,
        grid=(N_INST, NB, NH),
        in_specs=[
            pl.BlockSpec((1, B_BLK, D_IN), lambda i, b, h: (i, b, 0)),
            pl.BlockSpec((1, H_BLK, D_IN), lambda i, b, h: (i, h, 0)),
            pl.BlockSpec((1, 1, H_BLK), lambda i, b, h: (i, 0, h)),
        ],
        out_specs=pl.BlockSpec((B_BLK, H_BLK), lambda i, b, h: (b, i * NH + h)),
        out_shape=jax.ShapeDtypeStruct((BATCH, N_INST * D_SAE), jnp.int32),
    )(xT, W_enc, b_enc3)


@functools.cache
def _sc_kernels():
    mesh = plsc.VectorSubcoreMesh(core_axis_name="c", subcore_axis_name="s",
                                  num_cores=NC, num_subcores=NS)

    def _zero_hist(hist):
        def zbody(j, _):
            hist[pl.ds(j * 16, 16)] = jnp.zeros((16,), jnp.int32)
            return 0
        lax.fori_loop(0, NBINS // 16, zbody, 0, unroll=8)

    def _run_chunks(acts_hbm, wid, bufs, sems, process_chunk):
        row0 = wid * (N_PER_TILE // ROW_W)

        def src(c):
            return acts_hbm.at[row0 + c // CPR, pl.ds((c % CPR) * CHUNK,
                                                      CHUNK)]

        for p in range(2):
            pltpu.make_async_copy(src(p), bufs[p], sems[p]).start()

        def outer(c2, _):
            for p in range(2):
                c = c2 * 2 + p
                buf, sem = bufs[p], sems[p]
                pltpu.make_async_copy(src(c), buf, sem).wait()
                process_chunk(buf)

                @pl.when(c + 2 < NCHUNKS)
                def _():
                    pltpu.make_async_copy(src(c + 2), buf, sem).start()
            return 0
        lax.fori_loop(0, NCHUNKS // 2, outer, 0)

    @functools.partial(
        pl.kernel,
        out_type=jax.ShapeDtypeStruct((NTILES, NBINS), jnp.int32),
        mesh=mesh,
        compiler_params=pltpu.CompilerParams(needs_layout_passes=False),
        scratch_types=[
            pltpu.VMEM((NBINS,), jnp.int32),
            pltpu.VMEM((CHUNK,), jnp.int32),
            pltpu.VMEM((CHUNK,), jnp.int32),
            pltpu.SemaphoreType.DMA,
            pltpu.SemaphoreType.DMA,
        ],
    )
    def _hist_hi(acts_hbm, out_hbm, hist, buf0, buf1, sem0, sem1):
        wid = lax.axis_index("s") * NC + lax.axis_index("c")
        _zero_hist(hist)
        ones = jnp.full((16,), 1, jnp.int32)

        def process(buf):
            @plsc.parallel_loop(0, CHUNK, 16, unroll=UNROLL)
            def _(j):
                bits = buf[pl.ds(j, 16)]
                hi = lax.shift_right_logical(bits, 16)
                plsc.addupdate_scatter(hist, [hi], ones)

        _run_chunks(acts_hbm, wid, (buf0, buf1), (sem0, sem1), process)
        pltpu.sync_copy(hist, out_hbm.at[wid])

    @functools.partial(
        pl.kernel,
        out_type=jax.ShapeDtypeStruct((NTILES, NBINS), jnp.int32),
        mesh=mesh,
        compiler_params=pltpu.CompilerParams(needs_layout_passes=False),
        scratch_types=[
            pltpu.VMEM((NBINS,), jnp.int32),
            pltpu.VMEM((CHUNK,), jnp.int32),
            pltpu.VMEM((CHUNK,), jnp.int32),
            pltpu.VMEM((16,), jnp.int32),
            pltpu.SemaphoreType.DMA,
            pltpu.SemaphoreType.DMA,
        ],
    )
    def _hist_lo(acts_hbm, b1_hbm, out_hbm, hist, buf0, buf1, b1v,
                 sem0, sem1):
        wid = lax.axis_index("s") * NC + lax.axis_index("c")
        _zero_hist(hist)
        pltpu.sync_copy(b1_hbm, b1v)
        b1 = b1v[...]
        ones = jnp.full((16,), 1, jnp.int32)
        mask16 = jnp.full((16,), 0xFFFF, jnp.int32)

        def process(buf):
            @plsc.parallel_loop(0, CHUNK, 16, unroll=UNROLL)
            def _(j):
                bits = buf[pl.ds(j, 16)]
                hi = lax.shift_right_logical(bits, 16)
                lo = lax.bitwise_and(bits, mask16)
                plsc.addupdate_scatter(hist, [lo], ones, mask=hi == b1)

        _run_chunks(acts_hbm, wid, (buf0, buf1), (sem0, sem1), process)
        pltpu.sync_copy(hist, out_hbm.at[wid])

    return _hist_hi, _hist_lo


def _findbin_body(parts_ref, k_ref, bin_ref, kp_ref):
    p = parts_ref[...]
    idx = lax.broadcasted_iota(jnp.int32, p.shape, 1)
    k = k_ref[0, 0]

    def count_ge(m):
        return jnp.sum(jnp.where(idx >= m, p, 0))

    def step(_, carry):
        lo, hi = carry
        mid = (lo + hi + 1) // 2
        ge = count_ge(mid) >= k
        return (jnp.where(ge, mid, lo), jnp.where(ge, hi, mid - 1))

    lo, _ = lax.fori_loop(0, 16, step,
                          (jnp.int32(0), jnp.int32(NBINS - 1)))
    hb = jnp.sum(jnp.where(idx == lo, p, 0))
    bin_ref[0, 0] = lo
    kp_ref[0, 0] = k - (count_ge(lo) - hb)


def _findbin(parts, ktgt):
    return pl.pallas_call(
        _findbin_body,
        in_specs=[
            pl.BlockSpec(memory_space=pltpu.VMEM),
            pl.BlockSpec(memory_space=pltpu.SMEM),
        ],
        out_specs=[
            pl.BlockSpec(memory_space=pltpu.SMEM),
            pl.BlockSpec(memory_space=pltpu.SMEM),
        ],
        out_shape=[
            jax.ShapeDtypeStruct((1, 1), jnp.int32),
            jax.ShapeDtypeStruct((1, 1), jnp.int32),
        ],
    )(parts, ktgt)


def _dec_body(t_ref, acts_ref, w_ref, b_ref, topk_ref, topk2_ref,
              recon_ref):
    t = t_ref[0, 0]
    a = lax.bitcast_convert_type(acts_ref[...], jnp.float32)
    m = jnp.where(a >= t, a, 0.0)
    topk_ref[...] = m
    topk2_ref[...] = m
    contrib = lax.dot_general(m, w_ref[0], (((1,), (1,)), ((), ())),
                              preferred_element_type=jnp.float32)

    @pl.when(pl.program_id(2) == 0)
    def _():
        recon_ref[...] = jnp.zeros_like(recon_ref)

    recon_ref[...] += contrib[None]

    @pl.when(pl.program_id(2) == NH - 1)
    def _():
        recon_ref[...] = jnp.maximum(recon_ref[...] + b_ref[...], 0.0)


def _decode(t, acts2d, W_dec, b_dec3):
    return pl.pallas_call(
        _dec_body,
        grid=(N_INST, NB, NH),
        in_specs=[
            pl.BlockSpec(memory_space=pltpu.SMEM),
            pl.BlockSpec((B_BLK, H_BLK), lambda i, b, h: (b, i * NH + h)),
            pl.BlockSpec((1, D_IN, H_BLK), lambda i, b, h: (i, 0, h)),
            pl.BlockSpec((1, 1, D_IN), lambda i, b, h: (i, 0, 0)),
        ],
        out_specs=[
            pl.BlockSpec((B_BLK, H_BLK), lambda i, b, h: (b, i * NH + h)),
            pl.BlockSpec((B_BLK, H_BLK), lambda i, b, h: (b, i * NH + h)),
            pl.BlockSpec((1, B_BLK, D_IN), lambda i, b, h: (i, b, 0)),
        ],
        out_shape=[
            jax.ShapeDtypeStruct((BATCH, N_INST * D_SAE), jnp.float32),
            jax.ShapeDtypeStruct((BATCH, N_INST * D_SAE), jnp.float32),
            jax.ShapeDtypeStruct((N_INST, BATCH, D_IN), jnp.float32),
        ],
    )(t, acts2d, W_dec, b_dec3)


def kernel(x, W_enc, W_dec, b_enc, b_dec):
    xT = jnp.transpose(x.reshape(BATCH, N_INST, D_IN), (1, 0, 2))
    acts2d = _encode(xT, W_enc, b_enc.reshape(N_INST, 1, D_SAE))

    hist_hi, hist_lo = _sc_kernels()
    parts1 = hist_hi(acts2d)
    ktgt = jnp.full((1, 1), K_TOTAL, jnp.int32)
    b1, kp = _findbin(parts1, ktgt)

    b1vec = jnp.full((16,), 1, jnp.int32) * b1[0, 0]
    parts2 = hist_lo(acts2d, b1vec)
    b2, _ = _findbin(parts2, kp)

    t_bits = (b1[0, 0] << 16) | b2[0, 0]
    t = lax.bitcast_convert_type(t_bits, jnp.float32).reshape(1, 1)

    topk2d, topk2d_b, recon = _decode(t, acts2d, W_dec,
                                      b_dec.reshape(N_INST, 1, D_IN))
    recon_t = jnp.transpose(recon, (1, 0, 2))
    return (recon_t[None],
            topk2d.reshape(1, BATCH, N_INST, D_SAE),
            topk2d_b.reshape(BATCH, N_INST, D_SAE))


# speedup vs baseline: 1.2826x; 1.2826x over previous
import functools

import jax
import jax.numpy as jnp
from jax import lax
from jax.experimental import pallas as pl
from jax.experimental.pallas import tpu as pltpu
from jax.experimental.pallas import tpu_sc as plsc

BATCH = 1024
N_INST = 2
D_IN = 128
D_SAE = 16384
TOPK = 64
K_TOTAL = BATCH * N_INST * TOPK
N_TOTAL = BATCH * N_INST * D_SAE
NBINS = 65536

NC = 2
NS = 16
NTILES = NC * NS
N_PER_TILE = N_TOTAL // NTILES
CHUNK = 16384
NCHUNKS = N_PER_TILE // CHUNK
ROW_W = N_INST * D_SAE
CPR = ROW_W // CHUNK
UNROLL = 8

B_BLK = 256
H_BLK = 2048
NB = BATCH // B_BLK
NH = D_SAE // H_BLK


def _enc_body(x_ref, w_ref, b_ref, out_ref):
    a = lax.dot_general(x_ref[0], w_ref[0], (((1,), (1,)), ((), ())),
                        preferred_element_type=jnp.float32)
    out_ref[...] = lax.bitcast_convert_type(jnp.maximum(a + b_ref[0], 0.0),
                                            jnp.int32)


def _encode(xT, W_enc, b_enc3):
    return pl.pallas_call(
        _enc_body,
        grid=(N_INST, NB, NH),
        in_specs=[
            pl.BlockSpec((1, B_BLK, D_IN), lambda i, b, h: (i, b, 0)),
            pl.BlockSpec((1, H_BLK, D_IN), lambda i, b, h: (i, h, 0)),
            pl.BlockSpec((1, 1, H_BLK), lambda i, b, h: (i, 0, h)),
        ],
        out_specs=pl.BlockSpec((B_BLK, H_BLK), lambda i, b, h: (b, i * NH + h)),
        out_shape=jax.ShapeDtypeStruct((BATCH, N_INST * D_SAE), jnp.int32),
    )(xT, W_enc, b_enc3)


@functools.cache
def _sc_kernels():
    mesh = plsc.VectorSubcoreMesh(core_axis_name="c", subcore_axis_name="s",
                                  num_cores=NC, num_subcores=NS)

    def _zero_hist(hist):
        @plsc.parallel_loop(0, NBINS, 16, unroll=8)
        def _(j):
            hist[pl.ds(j, 16)] = jnp.zeros((16,), jnp.int32)

    def _run_chunks(acts_hbm, wid, bufs, sems, process_chunk):
        row0 = wid * (N_PER_TILE // ROW_W)

        def src(c):
            return acts_hbm.at[row0 + c // CPR, pl.ds((c % CPR) * CHUNK,
                                                      CHUNK)]

        for p in range(2):
            pltpu.make_async_copy(src(p), bufs[p], sems[p]).start()

        def outer(c2, _):
            for p in range(2):
                c = c2 * 2 + p
                buf, sem = bufs[p], sems[p]
                pltpu.make_async_copy(src(c), buf, sem).wait()
                process_chunk(buf)

                @pl.when(c + 2 < NCHUNKS)
                def _():
                    pltpu.make_async_copy(src(c + 2), buf, sem).start()
            return 0
        lax.fori_loop(0, NCHUNKS // 2, outer, 0)

    @functools.partial(
        pl.kernel,
        out_type=jax.ShapeDtypeStruct((NTILES, NBINS), jnp.int32),
        mesh=mesh,
        compiler_params=pltpu.CompilerParams(needs_layout_passes=False),
        scratch_types=[
            pltpu.VMEM((NBINS,), jnp.int32),
            pltpu.VMEM((CHUNK,), jnp.int32),
            pltpu.VMEM((CHUNK,), jnp.int32),
            pltpu.SemaphoreType.DMA,
            pltpu.SemaphoreType.DMA,
        ],
    )
    def _hist_hi(acts_hbm, out_hbm, hist, buf0, buf1, sem0, sem1):
        wid = lax.axis_index("s") * NC + lax.axis_index("c")
        _zero_hist(hist)
        ones = jnp.full((16,), 1, jnp.int32)

        def process(buf):
            @plsc.parallel_loop(0, CHUNK, 16, unroll=UNROLL)
            def _(j):
                bits = buf[pl.ds(j, 16)]
                hi = lax.shift_right_logical(bits, 16)
                plsc.addupdate_scatter(hist, [hi], ones)

        _run_chunks(acts_hbm, wid, (buf0, buf1), (sem0, sem1), process)
        pltpu.sync_copy(hist, out_hbm.at[wid])

    @functools.partial(
        pl.kernel,
        out_type=jax.ShapeDtypeStruct((NTILES, NBINS), jnp.int32),
        mesh=mesh,
        compiler_params=pltpu.CompilerParams(needs_layout_passes=False),
        scratch_types=[
            pltpu.VMEM((NBINS,), jnp.int32),
            pltpu.VMEM((CHUNK,), jnp.int32),
        ],
    )
    def _hist_sample(acts_hbm, out_hbm, hist, buf):
        wid = lax.axis_index("s") * NC + lax.axis_index("c")
        _zero_hist(hist)
        ones = jnp.full((16,), 1, jnp.int32)
        row0 = wid * (N_PER_TILE // ROW_W)
        pltpu.sync_copy(acts_hbm.at[row0, pl.ds(0, CHUNK)], buf)

        @plsc.parallel_loop(0, CHUNK, 16, unroll=UNROLL)
        def _(j):
            bits = buf[pl.ds(j, 16)]
            hi = lax.shift_right_logical(bits, 16)
            plsc.addupdate_scatter(hist, [hi], ones)

        pltpu.sync_copy(hist, out_hbm.at[wid])

    @functools.partial(
        pl.kernel,
        out_type=jax.ShapeDtypeStruct((NTILES, NBINS), jnp.int32),
        mesh=mesh,
        compiler_params=pltpu.CompilerParams(needs_layout_passes=False),
        scratch_types=[
            pltpu.VMEM((NBINS,), jnp.int32),
            pltpu.VMEM((CHUNK,), jnp.int32),
            pltpu.VMEM((CHUNK,), jnp.int32),
            pltpu.VMEM((16,), jnp.int32),
            pltpu.SemaphoreType.DMA,
            pltpu.SemaphoreType.DMA,
        ],
    )
    def _hist_hi_masked(acts_hbm, blow_hbm, out_hbm, hist, buf0, buf1,
                        blowv, sem0, sem1):
        wid = lax.axis_index("s") * NC + lax.axis_index("c")
        _zero_hist(hist)
        pltpu.sync_copy(blow_hbm, blowv)
        blow = blowv[...]
        ones = jnp.full((16,), 1, jnp.int32)

        def process(buf):
            @plsc.parallel_loop(0, CHUNK, 16, unroll=UNROLL)
            def _(j):
                bits = buf[pl.ds(j, 16)]
                hi = lax.shift_right_logical(bits, 16)
                plsc.addupdate_scatter(hist, [hi], ones, mask=hi >= blow)

        _run_chunks(acts_hbm, wid, (buf0, buf1), (sem0, sem1), process)
        pltpu.sync_copy(hist, out_hbm.at[wid])

    @functools.partial(
        pl.kernel,
        out_type=jax.ShapeDtypeStruct((NTILES, NBINS), jnp.int32),
        mesh=mesh,
        compiler_params=pltpu.CompilerParams(needs_layout_passes=False),
        scratch_types=[
            pltpu.VMEM((NBINS,), jnp.int32),
            pltpu.VMEM((CHUNK,), jnp.int32),
            pltpu.VMEM((CHUNK,), jnp.int32),
            pltpu.VMEM((16,), jnp.int32),
            pltpu.SemaphoreType.DMA,
            pltpu.SemaphoreType.DMA,
        ],
    )
    def _hist_lo(acts_hbm, b1_hbm, out_hbm, hist, buf0, buf1, b1v,
                 sem0, sem1):
        wid = lax.axis_index("s") * NC + lax.axis_index("c")
        _zero_hist(hist)
        pltpu.sync_copy(b1_hbm, b1v)
        b1 = b1v[...]
        ones = jnp.full((16,), 1, jnp.int32)
        mask16 = jnp.full((16,), 0xFFFF, jnp.int32)

        def process(buf):
            @plsc.parallel_loop(0, CHUNK, 16, unroll=UNROLL)
            def _(j):
                bits = buf[pl.ds(j, 16)]
                hi = lax.shift_right_logical(bits, 16)
                lo = lax.bitwise_and(bits, mask16)
                plsc.addupdate_scatter(hist, [lo], ones, mask=hi == b1)

        _run_chunks(acts_hbm, wid, (buf0, buf1), (sem0, sem1), process)
        pltpu.sync_copy(hist, out_hbm.at[wid])

    return _hist_hi, _hist_lo, _hist_sample, _hist_hi_masked


def _findbin_body(parts_ref, k_ref, bin_ref, kp_ref, tot_ref):
    p = parts_ref[...]
    idx = lax.broadcasted_iota(jnp.int32, p.shape, 1)
    k = k_ref[0, 0]

    def count_ge(m):
        return jnp.sum(jnp.where(idx >= m, p, 0))

    def step(_, carry):
        lo, hi = carry
        mid = (lo + hi + 1) // 2
        ge = count_ge(mid) >= k
        return (jnp.where(ge, mid, lo), jnp.where(ge, hi, mid - 1))

    lo, _ = lax.fori_loop(0, 16, step,
                          (jnp.int32(0), jnp.int32(NBINS - 1)))
    hb = jnp.sum(jnp.where(idx == lo, p, 0))
    bin_ref[0, 0] = lo
    kp_ref[0, 0] = k - (count_ge(lo) - hb)
    tot_ref[0, 0] = jnp.sum(p)


def _findbin(parts, ktgt):
    return pl.pallas_call(
        _findbin_body,
        in_specs=[
            pl.BlockSpec(memory_space=pltpu.VMEM),
            pl.BlockSpec(memory_space=pltpu.SMEM),
        ],
        out_specs=[
            pl.BlockSpec(memory_space=pltpu.SMEM),
            pl.BlockSpec(memory_space=pltpu.SMEM),
            pl.BlockSpec(memory_space=pltpu.SMEM),
        ],
        out_shape=[
            jax.ShapeDtypeStruct((1, 1), jnp.int32),
            jax.ShapeDtypeStruct((1, 1), jnp.int32),
            jax.ShapeDtypeStruct((1, 1), jnp.int32),
        ],
    )(parts, ktgt)


def _dec_body(t_ref, acts_ref, w_ref, b_ref, topk_ref, recon_ref):
    t = t_ref[0, 0]
    a = lax.bitcast_convert_type(acts_ref[...], jnp.float32)
    m = jnp.where(a >= t, a, 0.0)
    topk_ref[...] = m
    contrib = lax.dot_general(m, w_ref[0], (((1,), (1,)), ((), ())),
                              preferred_element_type=jnp.float32)

    @pl.when(pl.program_id(2) == 0)
    def _():
        recon_ref[...] = jnp.zeros_like(recon_ref)

    recon_ref[...] += contrib[None]

    @pl.when(pl.program_id(2) == NH - 1)
    def _():
        recon_ref[...] = jnp.maximum(recon_ref[...] + b_ref[...], 0.0)


def _decode(t, acts2d, W_dec, b_dec3):
    return pl.pallas_call(
        _dec_body,
        grid=(N_INST, NB, NH),
        in_specs=[
            pl.BlockSpec(memory_space=pltpu.SMEM),
            pl.BlockSpec((B_BLK, H_BLK), lambda i, b, h: (b, i * NH + h)),
            pl.BlockSpec((1, D_IN, H_BLK), lambda i, b, h: (i, 0, h)),
            pl.BlockSpec((1, 1, D_IN), lambda i, b, h: (i, 0, 0)),
        ],
        out_specs=[
            pl.BlockSpec((B_BLK, H_BLK), lambda i, b, h: (b, i * NH + h)),
            pl.BlockSpec((1, B_BLK, D_IN), lambda i, b, h: (i, b, 0)),
        ],
        out_shape=[
            jax.ShapeDtypeStruct((BATCH, N_INST * D_SAE), jnp.float32),
            jax.ShapeDtypeStruct((N_INST, BATCH, D_IN), jnp.float32),
        ],
    )(t, acts2d, W_dec, b_dec3)


def kernel(x, W_enc, W_dec, b_enc, b_dec):
    xT = jnp.transpose(x.reshape(BATCH, N_INST, D_IN), (1, 0, 2))
    acts2d = _encode(xT, W_enc, b_enc.reshape(N_INST, 1, D_SAE))

    hist_hi, hist_lo, hist_sample, hist_hi_masked = _sc_kernels()
    ktgt = jnp.full((1, 1), K_TOTAL, jnp.int32)

    starget = jnp.full((1, 1), 4 * K_TOTAL // 64, jnp.int32)
    parts0 = hist_sample(acts2d)
    blow, _, _ = _findbin(parts0, starget)
    blowvec = jnp.full((16,), 1, jnp.int32) * blow[0, 0]

    parts1 = hist_hi_masked(acts2d, blowvec)
    b1m, kpm, tot = _findbin(parts1, ktgt)

    def _good(a):
        return b1m, kpm

    def _bad(a):
        pf = hist_hi(a)
        bb, kk, _ = _findbin(pf, ktgt)
        return bb, kk

    b1, kp = lax.cond(tot[0, 0] >= K_TOTAL, _good, _bad, acts2d)

    b1vec = jnp.full((16,), 1, jnp.int32) * b1[0, 0]
    parts2 = hist_lo(acts2d, b1vec)
    b2, _, _ = _findbin(parts2, kp)

    t_bits = (b1[0, 0] << 16) | b2[0, 0]
    t = lax.bitcast_convert_type(t_bits, jnp.float32).reshape(1, 1)

    topk2d, recon = _decode(t, acts2d, W_dec,
                            b_dec.reshape(N_INST, 1, D_IN))
    topk3 = topk2d.reshape(BATCH, N_INST, D_SAE)
    recon_t = jnp.transpose(recon, (1, 0, 2))
    return (recon_t[None], topk3[None], topk3)


# speedup vs baseline: 1.2875x; 1.0038x over previous
import functools

import jax
import jax.numpy as jnp
from jax import lax
from jax.experimental import pallas as pl
from jax.experimental.pallas import tpu as pltpu
from jax.experimental.pallas import tpu_sc as plsc

BATCH = 1024
N_INST = 2
D_IN = 128
D_SAE = 16384
TOPK = 64
K_TOTAL = BATCH * N_INST * TOPK
N_TOTAL = BATCH * N_INST * D_SAE
NBINS = 65536

NC = 2
NS = 16
NTILES = NC * NS
N_PER_TILE = N_TOTAL // NTILES
CHUNK = 16384
NCHUNKS = N_PER_TILE // CHUNK
ROW_W = N_INST * D_SAE
CPR = ROW_W // CHUNK
UNROLL = 8

B_BLK = 256
H_BLK = 2048
NB = BATCH // B_BLK
NH = D_SAE // H_BLK


def _enc_body(x_ref, w_ref, b_ref, out_ref):
    a = lax.dot_general(x_ref[0], w_ref[0], (((1,), (1,)), ((), ())),
                        preferred_element_type=jnp.float32)
    out_ref[...] = lax.bitcast_convert_type(jnp.maximum(a + b_ref[0], 0.0),
                                            jnp.int32)


def _encode(xT, W_enc, b_enc3):
    return pl.pallas_call(
        _enc_body,
        grid=(N_INST, NB, NH),
        in_specs=[
            pl.BlockSpec((1, B_BLK, D_IN), lambda i, b, h: (i, b, 0)),
            pl.BlockSpec((1, H_BLK, D_IN), lambda i, b, h: (i, h, 0)),
            pl.BlockSpec((1, 1, H_BLK), lambda i, b, h: (i, 0, h)),
        ],
        out_specs=pl.BlockSpec((B_BLK, H_BLK), lambda i, b, h: (b, i * NH + h)),
        out_shape=jax.ShapeDtypeStruct((BATCH, N_INST * D_SAE), jnp.int32),
    )(xT, W_enc, b_enc3)


@functools.cache
def _sc_kernels():
    mesh = plsc.VectorSubcoreMesh(core_axis_name="c", subcore_axis_name="s",
                                  num_cores=NC, num_subcores=NS)

    def _zero_hist(hist):
        @plsc.parallel_loop(0, NBINS, 16, unroll=8)
        def _(j):
            hist[pl.ds(j, 16)] = jnp.zeros((16,), jnp.int32)

    def _run_chunks(acts_hbm, wid, bufs, sems, process_chunk):
        row0 = wid * (N_PER_TILE // ROW_W)

        def src(c):
            return acts_hbm.at[row0 + c // CPR, pl.ds((c % CPR) * CHUNK,
                                                      CHUNK)]

        for p in range(2):
            pltpu.make_async_copy(src(p), bufs[p], sems[p]).start()

        def outer(c2, _):
            for p in range(2):
                c = c2 * 2 + p
                buf, sem = bufs[p], sems[p]
                pltpu.make_async_copy(src(c), buf, sem).wait()
                process_chunk(buf)

                @pl.when(c + 2 < NCHUNKS)
                def _():
                    pltpu.make_async_copy(src(c + 2), buf, sem).start()
            return 0
        lax.fori_loop(0, NCHUNKS // 2, outer, 0)

    @functools.partial(
        pl.kernel,
        out_type=jax.ShapeDtypeStruct((NTILES, NBINS), jnp.int32),
        mesh=mesh,
        compiler_params=pltpu.CompilerParams(needs_layout_passes=False),
        scratch_types=[
            pltpu.VMEM((NBINS,), jnp.int32),
            pltpu.VMEM((CHUNK,), jnp.int32),
            pltpu.VMEM((CHUNK,), jnp.int32),
            pltpu.SemaphoreType.DMA,
            pltpu.SemaphoreType.DMA,
        ],
    )
    def _hist_hi(acts_hbm, out_hbm, hist, buf0, buf1, sem0, sem1):
        wid = lax.axis_index("s") * NC + lax.axis_index("c")
        _zero_hist(hist)
        ones = jnp.full((16,), 1, jnp.int32)

        def process(buf):
            @plsc.parallel_loop(0, CHUNK, 16, unroll=UNROLL)
            def _(j):
                bits = buf[pl.ds(j, 16)]
                hi = lax.shift_right_logical(bits, 16)
                plsc.addupdate_scatter(hist, [hi], ones)

        _run_chunks(acts_hbm, wid, (buf0, buf1), (sem0, sem1), process)
        pltpu.sync_copy(hist, out_hbm.at[wid])

    @functools.partial(
        pl.kernel,
        out_type=jax.ShapeDtypeStruct((NTILES, NBINS), jnp.int32),
        mesh=mesh,
        compiler_params=pltpu.CompilerParams(needs_layout_passes=False),
        scratch_types=[
            pltpu.VMEM((NBINS,), jnp.int32),
            pltpu.VMEM((CHUNK,), jnp.int32),
        ],
    )
    def _hist_sample(acts_hbm, out_hbm, hist, buf):
        wid = lax.axis_index("s") * NC + lax.axis_index("c")
        _zero_hist(hist)
        ones = jnp.full((16,), 1, jnp.int32)
        row0 = wid * (N_PER_TILE // ROW_W)
        pltpu.sync_copy(acts_hbm.at[row0, pl.ds(0, CHUNK)], buf)

        @plsc.parallel_loop(0, CHUNK, 16, unroll=UNROLL)
        def _(j):
            bits = buf[pl.ds(j, 16)]
            hi = lax.shift_right_logical(bits, 16)
            plsc.addupdate_scatter(hist, [hi], ones)

        pltpu.sync_copy(hist, out_hbm.at[wid])

    @functools.partial(
        pl.kernel,
        out_type=jax.ShapeDtypeStruct((NTILES, NBINS), jnp.int32),
        mesh=mesh,
        compiler_params=pltpu.CompilerParams(needs_layout_passes=False),
        scratch_types=[
            pltpu.VMEM((NBINS,), jnp.int32),
            pltpu.VMEM((CHUNK,), jnp.int32),
            pltpu.VMEM((CHUNK,), jnp.int32),
            pltpu.VMEM((16,), jnp.int32),
            pltpu.SemaphoreType.DMA,
            pltpu.SemaphoreType.DMA,
        ],
    )
    def _hist_hi_masked(acts_hbm, blow_hbm, out_hbm, hist, buf0, buf1,
                        blowv, sem0, sem1):
        wid = lax.axis_index("s") * NC + lax.axis_index("c")
        _zero_hist(hist)
        pltpu.sync_copy(blow_hbm, blowv)
        blow = blowv[...]
        ones = jnp.full((16,), 1, jnp.int32)

        def process(buf):
            @plsc.parallel_loop(0, CHUNK, 16, unroll=UNROLL)
            def _(j):
                bits = buf[pl.ds(j, 16)]
                hi = lax.shift_right_logical(bits, 16)
                plsc.addupdate_scatter(hist, [hi], ones, mask=hi >= blow)

        _run_chunks(acts_hbm, wid, (buf0, buf1), (sem0, sem1), process)
        pltpu.sync_copy(hist, out_hbm.at[wid])

    @functools.partial(
        pl.kernel,
        out_type=jax.ShapeDtypeStruct((NTILES, NBINS), jnp.int32),
        mesh=mesh,
        compiler_params=pltpu.CompilerParams(needs_layout_passes=False),
        scratch_types=[
            pltpu.VMEM((NBINS,), jnp.int32),
            pltpu.VMEM((CHUNK,), jnp.int32),
            pltpu.VMEM((CHUNK,), jnp.int32),
            pltpu.VMEM((16,), jnp.int32),
            pltpu.SemaphoreType.DMA,
            pltpu.SemaphoreType.DMA,
        ],
    )
    def _hist_lo(acts_hbm, b1_hbm, out_hbm, hist, buf0, buf1, b1v,
                 sem0, sem1):
        wid = lax.axis_index("s") * NC + lax.axis_index("c")
        _zero_hist(hist)
        pltpu.sync_copy(b1_hbm, b1v)
        b1 = b1v[...]
        ones = jnp.full((16,), 1, jnp.int32)
        mask16 = jnp.full((16,), 0xFFFF, jnp.int32)

        def process(buf):
            @plsc.parallel_loop(0, CHUNK, 16, unroll=UNROLL)
            def _(j):
                bits = buf[pl.ds(j, 16)]
                hi = lax.shift_right_logical(bits, 16)
                lo = lax.bitwise_and(bits, mask16)
                plsc.addupdate_scatter(hist, [lo], ones, mask=hi == b1)

        _run_chunks(acts_hbm, wid, (buf0, buf1), (sem0, sem1), process)
        pltpu.sync_copy(hist, out_hbm.at[wid])

    return _hist_hi, _hist_lo, _hist_sample, _hist_hi_masked


def _search(p, k):
    idx = lax.broadcasted_iota(jnp.int32, p.shape, 1)

    def count_ge(m):
        return jnp.sum(jnp.where(idx >= m, p, 0))

    def step(_, carry):
        lo, hi = carry
        mid = (lo + hi + 1) // 2
        ge = count_ge(mid) >= k
        return (jnp.where(ge, mid, lo), jnp.where(ge, hi, mid - 1))

    lo, _ = lax.fori_loop(0, 16, step,
                          (jnp.int32(0), jnp.int32(NBINS - 1)))
    hb = jnp.sum(jnp.where(idx == lo, p, 0))
    return lo, k - (count_ge(lo) - hb)


def _findbin_body(parts_ref, k_ref, bin_ref, binvec_ref, kp_ref, tot_ref):
    p = parts_ref[...]
    lo, kp = _search(p, k_ref[0, 0])
    bin_ref[0, 0] = lo
    binvec_ref[...] = jnp.full((16,), 1, jnp.int32) * lo
    kp_ref[0, 0] = kp
    tot_ref[0, 0] = jnp.sum(p)


def _findbin(parts, ktgt):
    return pl.pallas_call(
        _findbin_body,
        in_specs=[
            pl.BlockSpec(memory_space=pltpu.VMEM),
            pl.BlockSpec(memory_space=pltpu.SMEM),
        ],
        out_specs=[
            pl.BlockSpec(memory_space=pltpu.SMEM),
            pl.BlockSpec(memory_space=pltpu.VMEM),
            pl.BlockSpec(memory_space=pltpu.SMEM),
            pl.BlockSpec(memory_space=pltpu.SMEM),
        ],
        out_shape=[
            jax.ShapeDtypeStruct((1, 1), jnp.int32),
            jax.ShapeDtypeStruct((16,), jnp.int32),
            jax.ShapeDtypeStruct((1, 1), jnp.int32),
            jax.ShapeDtypeStruct((1, 1), jnp.int32),
        ],
    )(parts, ktgt)


def _findbin2_body(parts_ref, k_ref, b1_ref, t_ref):
    p = parts_ref[...]
    lo, _ = _search(p, k_ref[0, 0])
    t_bits = lax.shift_left(b1_ref[0, 0], 16) | lo
    t_ref[0, 0] = lax.bitcast_convert_type(t_bits, jnp.float32)


def _findbin2(parts, ktgt, b1):
    return pl.pallas_call(
        _findbin2_body,
        in_specs=[
            pl.BlockSpec(memory_space=pltpu.VMEM),
            pl.BlockSpec(memory_space=pltpu.SMEM),
            pl.BlockSpec(memory_space=pltpu.SMEM),
        ],
        out_specs=pl.BlockSpec(memory_space=pltpu.SMEM),
        out_shape=jax.ShapeDtypeStruct((1, 1), jnp.float32),
    )(parts, ktgt, b1)


def _dec_body(t_ref, acts_ref, w_ref, b_ref, topk_ref, recon_ref):
    t = t_ref[0, 0]
    a = lax.bitcast_convert_type(acts_ref[...], jnp.float32)
    m = jnp.where(a >= t, a, 0.0)
    topk_ref[...] = m
    contrib = lax.dot_general(m, w_ref[0], (((1,), (1,)), ((), ())),
                              preferred_element_type=jnp.float32)

    @pl.when(pl.program_id(2) == 0)
    def _():
        recon_ref[...] = jnp.zeros_like(recon_ref)

    recon_ref[...] += contrib[None]

    @pl.when(pl.program_id(2) == NH - 1)
    def _():
        recon_ref[...] = jnp.maximum(recon_ref[...] + b_ref[...], 0.0)


def _decode(t, acts2d, W_dec, b_dec3):
    return pl.pallas_call(
        _dec_body,
        grid=(N_INST, NB, NH),
        in_specs=[
            pl.BlockSpec(memory_space=pltpu.SMEM),
            pl.BlockSpec((B_BLK, H_BLK), lambda i, b, h: (b, i * NH + h)),
            pl.BlockSpec((1, D_IN, H_BLK), lambda i, b, h: (i, 0, h)),
            pl.BlockSpec((1, 1, D_IN), lambda i, b, h: (i, 0, 0)),
        ],
        out_specs=[
            pl.BlockSpec((B_BLK, H_BLK), lambda i, b, h: (b, i * NH + h)),
            pl.BlockSpec((1, B_BLK, D_IN), lambda i, b, h: (i, b, 0)),
        ],
        out_shape=[
            jax.ShapeDtypeStruct((BATCH, N_INST * D_SAE), jnp.float32),
            jax.ShapeDtypeStruct((N_INST, BATCH, D_IN), jnp.float32),
        ],
    )(t, acts2d, W_dec, b_dec3)


def kernel(x, W_enc, W_dec, b_enc, b_dec):
    xT = jnp.transpose(x.reshape(BATCH, N_INST, D_IN), (1, 0, 2))
    acts2d = _encode(xT, W_enc, b_enc.reshape(N_INST, 1, D_SAE))

    hist_hi, hist_lo, hist_sample, hist_hi_masked = _sc_kernels()
    ktgt = jnp.full((1, 1), K_TOTAL, jnp.int32)

    starget = jnp.full((1, 1), 4 * K_TOTAL // 64, jnp.int32)
    parts0 = hist_sample(acts2d)
    _, blowvec, _, _ = _findbin(parts0, starget)

    parts1 = hist_hi_masked(acts2d, blowvec)
    b1m, b1vecm, kpm, tot = _findbin(parts1, ktgt)

    def _good(a):
        return b1m, b1vecm, kpm

    def _bad(a):
        pf = hist_hi(a)
        bb, bv, kk, _ = _findbin(pf, ktgt)
        return bb, bv, kk

    b1, b1vec, kp = lax.cond(tot[0, 0] >= K_TOTAL, _good, _bad, acts2d)

    parts2 = hist_lo(acts2d, b1vec)
    t = _findbin2(parts2, kp, b1)

    topk2d, recon = _decode(t, acts2d, W_dec,
                            b_dec.reshape(N_INST, 1, D_IN))
    topk3 = topk2d.reshape(BATCH, N_INST, D_SAE)
    recon_t = jnp.transpose(recon, (1, 0, 2))
    return (recon_t[None], topk3[None], topk3)


# speedup vs baseline: 1.3585x; 1.0551x over previous
import functools

import jax
import jax.numpy as jnp
from jax import lax
from jax.experimental import pallas as pl
from jax.experimental.pallas import tpu as pltpu
from jax.experimental.pallas import tpu_sc as plsc

BATCH = 1024
N_INST = 2
D_IN = 128
D_SAE = 16384
TOPK = 64
K_TOTAL = BATCH * N_INST * TOPK
N_TOTAL = BATCH * N_INST * D_SAE
NBINS = 65536

NC = 2
NS = 16
NTILES = NC * NS
N_PER_TILE = N_TOTAL // NTILES
CHUNK = 16384
NCHUNKS = N_PER_TILE // CHUNK
ROW_W = N_INST * D_SAE
CPR = ROW_W // CHUNK
UNROLL = 8

B_BLK = 256
H_BLK = 2048
NB = BATCH // B_BLK
NH = D_SAE // H_BLK


def _enc_body(x_ref, w_ref, b_ref, out_ref):
    a = lax.dot_general(x_ref[0], w_ref[0], (((1,), (1,)), ((), ())),
                        preferred_element_type=jnp.float32)
    out_ref[...] = lax.bitcast_convert_type(jnp.maximum(a + b_ref[0], 0.0),
                                            jnp.int32)


def _encode(xT, W_enc, b_enc3):
    return pl.pallas_call(
        _enc_body,
        grid=(N_INST, NB, NH),
        in_specs=[
            pl.BlockSpec((1, B_BLK, D_IN), lambda i, b, h: (i, b, 0)),
            pl.BlockSpec((1, H_BLK, D_IN), lambda i, b, h: (i, h, 0)),
            pl.BlockSpec((1, 1, H_BLK), lambda i, b, h: (i, 0, h)),
        ],
        out_specs=pl.BlockSpec((B_BLK, H_BLK), lambda i, b, h: (b, i * NH + h)),
        out_shape=jax.ShapeDtypeStruct((BATCH, N_INST * D_SAE), jnp.int32),
    )(xT, W_enc, b_enc3)


@functools.cache
def _sc_kernels():
    mesh = plsc.VectorSubcoreMesh(core_axis_name="c", subcore_axis_name="s",
                                  num_cores=NC, num_subcores=NS)

    def _zero_hist(hist):
        @plsc.parallel_loop(0, NBINS, 16, unroll=8)
        def _(j):
            hist[pl.ds(j, 16)] = jnp.zeros((16,), jnp.int32)

    def _run_chunks(acts_hbm, wid, bufs, sems, process_chunk):
        row0 = wid * (N_PER_TILE // ROW_W)

        def src(c):
            return acts_hbm.at[row0 + c // CPR, pl.ds((c % CPR) * CHUNK,
                                                      CHUNK)]

        for p in range(2):
            pltpu.make_async_copy(src(p), bufs[p], sems[p]).start()

        def outer(c2, _):
            for p in range(2):
                c = c2 * 2 + p
                buf, sem = bufs[p], sems[p]
                pltpu.make_async_copy(src(c), buf, sem).wait()
                process_chunk(buf)

                @pl.when(c + 2 < NCHUNKS)
                def _():
                    pltpu.make_async_copy(src(c + 2), buf, sem).start()
            return 0
        lax.fori_loop(0, NCHUNKS // 2, outer, 0)

    @functools.partial(
        pl.kernel,
        out_type=jax.ShapeDtypeStruct((NTILES, NBINS), jnp.int32),
        mesh=mesh,
        compiler_params=pltpu.CompilerParams(needs_layout_passes=False),
        scratch_types=[
            pltpu.VMEM((NBINS,), jnp.int32),
            pltpu.VMEM((CHUNK,), jnp.int32),
            pltpu.VMEM((CHUNK,), jnp.int32),
            pltpu.SemaphoreType.DMA,
            pltpu.SemaphoreType.DMA,
        ],
    )
    def _hist_hi(acts_hbm, out_hbm, hist, buf0, buf1, sem0, sem1):
        wid = lax.axis_index("s") * NC + lax.axis_index("c")
        _zero_hist(hist)
        ones = jnp.full((16,), 1, jnp.int32)

        def process(buf):
            @plsc.parallel_loop(0, CHUNK, 16, unroll=UNROLL)
            def _(j):
                bits = buf[pl.ds(j, 16)]
                hi = lax.shift_right_logical(bits, 16)
                plsc.addupdate_scatter(hist, [hi], ones)

        _run_chunks(acts_hbm, wid, (buf0, buf1), (sem0, sem1), process)
        pltpu.sync_copy(hist, out_hbm.at[wid])

    @functools.partial(
        pl.kernel,
        out_type=(jax.ShapeDtypeStruct((NTILES, NBINS), jnp.int32),
                  jax.ShapeDtypeStruct((NTILES, 16), jnp.int32)),
        mesh=mesh,
        compiler_params=pltpu.CompilerParams(needs_layout_passes=False),
        scratch_types=[
            pltpu.VMEM((NBINS,), jnp.int32),
            pltpu.VMEM((CHUNK,), jnp.int32),
            pltpu.VMEM((CHUNK,), jnp.int32),
            pltpu.VMEM((16,), jnp.int32),
            pltpu.SemaphoreType.DMA,
            pltpu.SemaphoreType.DMA,
        ],
    )
    def _hist_hi_auto(acts_hbm, out_hbm, blows_hbm, hist, buf0, buf1,
                      blowv, sem0, sem1):
        wid = lax.axis_index("s") * NC + lax.axis_index("c")
        _zero_hist(hist)
        ones = jnp.full((16,), 1, jnp.int32)
        row0 = wid * (N_PER_TILE // ROW_W)
        pltpu.sync_copy(acts_hbm.at[row0, pl.ds(0, CHUNK)], buf0)

        @plsc.parallel_loop(0, CHUNK, 16, unroll=UNROLL,
                            carry=jnp.zeros((16,), jnp.int32))
        def mx16(j, m):
            bits = buf0[pl.ds(j, 16)]
            hi = lax.shift_right_logical(bits, 16)
            plsc.addupdate_scatter(hist, [hi], ones)
            return jnp.maximum(m, hi)

        g0 = jnp.max(mx16) // 16

        def w_cond(s):
            return jnp.logical_and(s[1] < 256, s[0] >= 0)

        def w_body(s):
            g, cum = s
            return g - 1, cum + jnp.sum(hist[pl.ds(g * 16, 16)])

        g, _ = lax.while_loop(w_cond, w_body, (g0, jnp.int32(0)))
        blow = jnp.maximum(g + 1, 0) * 16
        blowvec = jnp.full((16,), 1, jnp.int32) * blow
        blowv[...] = blowvec
        pltpu.sync_copy(blowv, blows_hbm.at[wid])

        _zero_hist(hist)

        def process(buf):
            @plsc.parallel_loop(0, CHUNK, 16, unroll=UNROLL)
            def _(j):
                bits = buf[pl.ds(j, 16)]
                hi = lax.shift_right_logical(bits, 16)
                plsc.addupdate_scatter(hist, [hi], ones,
                                       mask=hi >= blowvec)

        _run_chunks(acts_hbm, wid, (buf0, buf1), (sem0, sem1), process)
        pltpu.sync_copy(hist, out_hbm.at[wid])

    @functools.partial(
        pl.kernel,
        out_type=jax.ShapeDtypeStruct((NTILES, NBINS), jnp.int32),
        mesh=mesh,
        compiler_params=pltpu.CompilerParams(needs_layout_passes=False),
        scratch_types=[
            pltpu.VMEM((NBINS,), jnp.int32),
            pltpu.VMEM((CHUNK,), jnp.int32),
            pltpu.VMEM((CHUNK,), jnp.int32),
            pltpu.VMEM((16,), jnp.int32),
            pltpu.SemaphoreType.DMA,
            pltpu.SemaphoreType.DMA,
        ],
    )
    def _hist_lo(acts_hbm, b1_hbm, out_hbm, hist, buf0, buf1, b1v,
                 sem0, sem1):
        wid = lax.axis_index("s") * NC + lax.axis_index("c")
        _zero_hist(hist)
        pltpu.sync_copy(b1_hbm, b1v)
        b1 = b1v[...]
        ones = jnp.full((16,), 1, jnp.int32)
        mask16 = jnp.full((16,), 0xFFFF, jnp.int32)

        def process(buf):
            @plsc.parallel_loop(0, CHUNK, 16, unroll=UNROLL)
            def _(j):
                bits = buf[pl.ds(j, 16)]
                hi = lax.shift_right_logical(bits, 16)
                lo = lax.bitwise_and(bits, mask16)
                plsc.addupdate_scatter(hist, [lo], ones, mask=hi == b1)

        _run_chunks(acts_hbm, wid, (buf0, buf1), (sem0, sem1), process)
        pltpu.sync_copy(hist, out_hbm.at[wid])

    return _hist_hi, _hist_lo, _hist_hi_auto


def _search(p, k):
    idx = lax.broadcasted_iota(jnp.int32, p.shape, 1)

    def count_ge(m):
        return jnp.sum(jnp.where(idx >= m, p, 0))

    def step(_, carry):
        lo, hi = carry
        mid = (lo + hi + 1) // 2
        ge = count_ge(mid) >= k
        return (jnp.where(ge, mid, lo), jnp.where(ge, hi, mid - 1))

    lo, _ = lax.fori_loop(0, 16, step,
                          (jnp.int32(0), jnp.int32(NBINS - 1)))
    hb = jnp.sum(jnp.where(idx == lo, p, 0))
    return lo, k - (count_ge(lo) - hb)


def _findbin_body(parts_ref, blows_ref, k_ref, bin_ref, binvec_ref,
                  kp_ref, ok_ref):
    p = parts_ref[...]
    k = k_ref[0, 0]
    lo, kp = _search(p, k)
    bin_ref[0, 0] = lo
    binvec_ref[...] = jnp.full((16,), 1, jnp.int32) * lo
    kp_ref[0, 0] = kp
    maxblow = jnp.max(blows_ref[...])
    idx = lax.broadcasted_iota(jnp.int32, p.shape, 1)
    c_mb = jnp.sum(jnp.where(idx >= maxblow, p, 0))
    ok_ref[0, 0] = (c_mb >= k).astype(jnp.int32)


def _findbin(parts, blows, ktgt):
    return pl.pallas_call(
        _findbin_body,
        in_specs=[
            pl.BlockSpec(memory_space=pltpu.VMEM),
            pl.BlockSpec(memory_space=pltpu.VMEM),
            pl.BlockSpec(memory_space=pltpu.SMEM),
        ],
        out_specs=[
            pl.BlockSpec(memory_space=pltpu.SMEM),
            pl.BlockSpec(memory_space=pltpu.VMEM),
            pl.BlockSpec(memory_space=pltpu.SMEM),
            pl.BlockSpec(memory_space=pltpu.SMEM),
        ],
        out_shape=[
            jax.ShapeDtypeStruct((1, 1), jnp.int32),
            jax.ShapeDtypeStruct((16,), jnp.int32),
            jax.ShapeDtypeStruct((1, 1), jnp.int32),
            jax.ShapeDtypeStruct((1, 1), jnp.int32),
        ],
    )(parts, blows, ktgt)


def _findbin2_body(parts_ref, k_ref, b1_ref, t_ref):
    p = parts_ref[...]
    lo, _ = _search(p, k_ref[0, 0])
    t_bits = lax.shift_left(b1_ref[0, 0], 16) | lo
    t_ref[0, 0] = lax.bitcast_convert_type(t_bits, jnp.float32)


def _findbin2(parts, ktgt, b1):
    return pl.pallas_call(
        _findbin2_body,
        in_specs=[
            pl.BlockSpec(memory_space=pltpu.VMEM),
            pl.BlockSpec(memory_space=pltpu.SMEM),
            pl.BlockSpec(memory_space=pltpu.SMEM),
        ],
        out_specs=pl.BlockSpec(memory_space=pltpu.SMEM),
        out_shape=jax.ShapeDtypeStruct((1, 1), jnp.float32),
    )(parts, ktgt, b1)


def _dec_body(t_ref, acts_ref, w_ref, b_ref, topk_ref, recon_ref):
    t = t_ref[0, 0]
    a = lax.bitcast_convert_type(acts_ref[...], jnp.float32)
    m = jnp.where(a >= t, a, 0.0)
    topk_ref[...] = m
    contrib = lax.dot_general(m, w_ref[0], (((1,), (1,)), ((), ())),
                              preferred_element_type=jnp.float32)

    @pl.when(pl.program_id(2) == 0)
    def _():
        recon_ref[...] = jnp.zeros_like(recon_ref)

    recon_ref[...] += contrib[None]

    @pl.when(pl.program_id(2) == NH - 1)
    def _():
        recon_ref[...] = jnp.maximum(recon_ref[...] + b_ref[...], 0.0)


def _decode(t, acts2d, W_dec, b_dec3):
    return pl.pallas_call(
        _dec_body,
        grid=(N_INST, NB, NH),
        in_specs=[
            pl.BlockSpec(memory_space=pltpu.SMEM),
            pl.BlockSpec((B_BLK, H_BLK), lambda i, b, h: (b, i * NH + h)),
            pl.BlockSpec((1, D_IN, H_BLK), lambda i, b, h: (i, 0, h)),
            pl.BlockSpec((1, 1, D_IN), lambda i, b, h: (i, 0, 0)),
        ],
        out_specs=[
            pl.BlockSpec((B_BLK, H_BLK), lambda i, b, h: (b, i * NH + h)),
            pl.BlockSpec((1, B_BLK, D_IN), lambda i, b, h: (i, b, 0)),
        ],
        out_shape=[
            jax.ShapeDtypeStruct((BATCH, N_INST * D_SAE), jnp.float32),
            jax.ShapeDtypeStruct((N_INST, BATCH, D_IN), jnp.float32),
        ],
    )(t, acts2d, W_dec, b_dec3)


def kernel(x, W_enc, W_dec, b_enc, b_dec):
    xT = jnp.transpose(x.reshape(BATCH, N_INST, D_IN), (1, 0, 2))
    acts2d = _encode(xT, W_enc, b_enc.reshape(N_INST, 1, D_SAE))

    hist_hi, hist_lo, hist_hi_auto = _sc_kernels()
    ktgt = jnp.full((1, 1), K_TOTAL, jnp.int32)

    parts1, blows = hist_hi_auto(acts2d)
    b1m, b1vecm, kpm, ok = _findbin(parts1, blows, ktgt)

    zblows = jnp.zeros((NTILES, 16), jnp.int32)

    def _good(a):
        return b1m, b1vecm, kpm

    def _bad(a):
        pf = hist_hi(a)
        bb, bv, kk, _ = _findbin(pf, zblows, ktgt)
        return bb, bv, kk

    b1, b1vec, kp = lax.cond(ok[0, 0] > 0, _good, _bad, acts2d)

    parts2 = hist_lo(acts2d, b1vec)
    t = _findbin2(parts2, kp, b1)

    topk2d, recon = _decode(t, acts2d, W_dec,
                            b_dec.reshape(N_INST, 1, D_IN))
    topk3 = topk2d.reshape(BATCH, N_INST, D_SAE)
    recon_t = jnp.transpose(recon, (1, 0, 2))
    return (recon_t[None], topk3[None], topk3)
